# baseline (speedup 1.0000x reference)
import functools

import jax
import jax.numpy as jnp
from jax import lax
from jax.experimental import pallas as pl
from jax.experimental.pallas import tpu as pltpu
from jax.experimental.pallas import tpu_sc as plsc

EMB_SIZE = 64
SCALE = 8.0
VOCAB = 1000000
CBLK = 128
N_FULL = VOCAB // CBLK


def _mesh():
    return plsc.VectorSubcoreMesh(core_axis_name="c", subcore_axis_name="s")


def _iota16():
    return lax.iota(jnp.int32, 16)


@functools.lru_cache(maxsize=None)
def _make_transpose_kernel():
    info = plsc.get_sparse_core_info()
    nc, ns = info.num_cores, info.num_subcores
    nw = nc * ns
    nbuf = 2
    n_rounds = -(-N_FULL // nw) + 1
    n_rounds += n_rounds % 2
    n_outer = n_rounds // 2

    @functools.partial(
        pl.kernel,
        mesh=_mesh(),
        out_type=jax.ShapeDtypeStruct((VOCAB // 2, CBLK), jnp.float32),
        compiler_params=pltpu.CompilerParams(use_tc_tiling_on_sc=True, needs_layout_passes=False),
        scratch_types=(
            [pltpu.VMEM((EMB_SIZE, CBLK), jnp.float32) for _ in range(2 * nbuf)]
            + [pltpu.SemaphoreType.DMA for _ in range(2 * nbuf)]
        ),
    )
    def sc_transpose(tab_hbm, tail_hbm, packed_hbm, *scratch):
        in_buf = scratch[:nbuf]
        out_buf = scratch[nbuf : 2 * nbuf]
        isem = scratch[2 * nbuf : 3 * nbuf]
        osem = scratch[3 * nbuf : 4 * nbuf]
        wid = lax.axis_index("s") * nc + lax.axis_index("c")

        @pl.when(wid == 0)
        def _():
            pltpu.sync_copy(tail_hbm, in_buf[0].at[pl.ds(0, 32)])
            pltpu.sync_copy(
                in_buf[0].at[pl.ds(0, 32)],
                packed_hbm.at[pl.ds(VOCAB // 2 - 32, 32)],
            )

        def slab(k):
            return jnp.minimum(wid + k * nw, N_FULL - 1)

        def load(k, b):
            return pltpu.make_async_copy(
                tab_hbm.at[:, pl.ds(slab(k) * CBLK, CBLK)], in_buf[b], isem[b]
            )

        def store(k, b):
            return pltpu.make_async_copy(
                out_buf[b],
                packed_hbm.at[pl.ds(slab(k) * (CBLK // 2), EMB_SIZE)],
                osem[b],
            )

        rowv = [_iota16() + 16 * g for g in range(4)]
        for b in range(nbuf):
            load(b, b).start()

        @pl.loop(0, n_outer)
        def outer(r):
            for b in range(nbuf):
                k = r * 2 + b
                load(k, b).wait()

                @pl.when(r > 0)
                def _():
                    store(k - 2, b).wait()

                @plsc.parallel_loop(0, EMB_SIZE)
                def transpose_row(u):
                    for h in range(2):
                        colv = jnp.full((16,), 0, jnp.int32) + (2 * u + h)
                        for g in range(4):
                            val = plsc.load_gather(in_buf[b], [rowv[g], colv])
                            out_buf[b][u, pl.ds(64 * h + 16 * g, 16)] = val * SCALE

                @pl.when(r < n_outer - 1)
                def _():
                    load(k + 2, b).start()

                store(k, b).start()

        for b in range(nbuf):
            store(n_outer * 2 - 2 + b, b).wait()

    return sc_transpose


@functools.lru_cache(maxsize=None)
def _make_gather_kernel(bsz: int, seq: int):
    info = plsc.get_sparse_core_info()
    nc, ns = info.num_cores, info.num_subcores
    nw = nc * ns
    assert bsz % (nw * CBLK) == 0 and seq % 2 == 0
    n_outer = seq // 2

    @functools.partial(
        pl.kernel,
        mesh=_mesh(),
        out_type=jax.ShapeDtypeStruct((seq, EMB_SIZE, bsz), jnp.float32),
        compiler_params=pltpu.CompilerParams(use_tc_tiling_on_sc=True, needs_layout_passes=False),
        scratch_types=(
            [
                pltpu.VMEM((seq, CBLK), jnp.int32),
                pltpu.VMEM((seq, CBLK), jnp.int32),
            ]
            + [pltpu.VMEM((CBLK, CBLK), jnp.float32) for _ in range(2)]
            + [pltpu.VMEM((EMB_SIZE, CBLK), jnp.float32) for _ in range(2)]
            + [pltpu.SemaphoreType.DMA for _ in range(4)]
        ),
    )
    def sc_gather(packed_hbm, tok_hbm, out_hbm, idx_v, half_v, *scratch):
        gbuf = scratch[:2]
        obuf = scratch[2:4]
        gsem = scratch[4:6]
        ssem = scratch[6:8]
        wid = lax.axis_index("s") * nc + lax.axis_index("c")
        col0 = wid * CBLK

        pltpu.sync_copy(tok_hbm.at[:, pl.ds(col0, CBLK)], idx_v)

        @pl.loop(0, seq)
        def halve(s):
            for m in range(8):
                sl = pl.ds(16 * m, 16)
                half_v[s, sl] = lax.shift_right_logical(idx_v[s, sl], 1)

        def gather(s, b):
            return pltpu.make_async_copy(
                packed_hbm.at[half_v.at[s]], gbuf[b], gsem[b]
            )

        def store(s, b):
            return pltpu.make_async_copy(
                obuf[b], out_hbm.at[s, :, pl.ds(col0, CBLK)], ssem[b]
            )

        rowv = [_iota16() + 16 * m for m in range(8)]
        for b in range(2):
            gather(b, b).start()

        @pl.loop(0, n_outer)
        def outer(r):
            for b in range(2):
                s = r * 2 + b
                gather(s, b).wait()

                @pl.when(r > 0)
                def _():
                    store(s - 2, b).wait()

                parv = [
                    lax.shift_left(
                        lax.bitwise_and(idx_v[s, pl.ds(16 * m, 16)], 1), 6
                    )
                    for m in range(8)
                ]

                @plsc.parallel_loop(0, EMB_SIZE)
                def reorder(e):
                    for m in range(8):
                        colv = parv[m] + e
                        val = plsc.load_gather(gbuf[b], [rowv[m], colv])
                        obuf[b][e, pl.ds(16 * m, 16)] = val

                @pl.when(r < n_outer - 1)
                def _():
                    gather(s + 2, b).start()

                store(s, b).start()

        for b in range(2):
            store(n_outer * 2 - 2 + b, b).wait()

    return sc_gather


@jax.jit
def kernel(tokens, emb_table):
    bsz, seq = tokens.shape
    tab_t = emb_table.T
    tail = (emb_table[VOCAB - 64 :] * SCALE).reshape(32, 2, 64).reshape(32, 128)
    packed = _make_transpose_kernel()(tab_t, tail)
    tok_t = tokens.astype(jnp.int32).T
    out_phys = _make_gather_kernel(bsz, seq)(packed, tok_t)
    return jnp.transpose(out_phys, (2, 0, 1))


# speedup vs baseline: 1.2571x; 1.2571x over previous
import functools

import jax
import jax.numpy as jnp
from jax import lax
from jax.experimental import pallas as pl
from jax.experimental.pallas import tpu as pltpu
from jax.experimental.pallas import tpu_sc as plsc

EMB_SIZE = 64
SCALE = 8.0
VOCAB = 1000000
CBLK = 128
GPITCH = 133


def _iota16():
    return lax.iota(jnp.int32, 16)


@functools.lru_cache(maxsize=None)
def _make_gather_kernel(bsz: int, seq: int):
    info = plsc.get_sparse_core_info()
    nc, ns = info.num_cores, info.num_subcores
    nw = nc * ns
    assert bsz % (nw * CBLK) == 0 and seq % 2 == 0
    n_outer = seq // 2
    mesh = plsc.VectorSubcoreMesh(core_axis_name="c", subcore_axis_name="s")

    @functools.partial(
        pl.kernel,
        mesh=mesh,
        out_type=jax.ShapeDtypeStruct((seq, EMB_SIZE, bsz), jnp.float32),
        compiler_params=pltpu.CompilerParams(
            use_tc_tiling_on_sc=True, needs_layout_passes=False
        ),
        scratch_types=(
            [
                pltpu.VMEM((seq, CBLK), jnp.int32),
                pltpu.VMEM((seq, CBLK), jnp.int32),
            ]
            + [pltpu.VMEM((CBLK, GPITCH), jnp.float32) for _ in range(2)]
            + [pltpu.VMEM((EMB_SIZE, CBLK), jnp.float32) for _ in range(2)]
            + [pltpu.SemaphoreType.DMA for _ in range(4)]
        ),
    )
    def sc_gather(packed_hbm, tok_hbm, out_hbm, idx_v, half_v, *scratch):
        gbuf = scratch[:2]
        obuf = scratch[2:4]
        gsem = scratch[4:6]
        ssem = scratch[6:8]
        wid = lax.axis_index("s") * nc + lax.axis_index("c")
        col0 = wid * CBLK

        pltpu.sync_copy(tok_hbm.at[:, pl.ds(col0, CBLK)], idx_v)

        @pl.loop(0, seq)
        def halve(s):
            for m in range(8):
                sl = pl.ds(16 * m, 16)
                half_v[s, sl] = lax.shift_right_logical(idx_v[s, sl], 1)

        def gather(s, b):
            return pltpu.make_async_copy(
                packed_hbm.at[half_v.at[s]],
                gbuf[b].at[:, pl.ds(0, 2 * EMB_SIZE)],
                gsem[b],
            )

        def store(s, b):
            return pltpu.make_async_copy(
                obuf[b], out_hbm.at[s, :, pl.ds(col0, CBLK)], ssem[b]
            )

        rowv = [_iota16() + 16 * m for m in range(8)]
        for b in range(2):
            gather(b, b).start()

        @pl.loop(0, n_outer)
        def outer(r):
            for b in range(2):
                s = r * 2 + b
                gather(s, b).wait()

                @pl.when(r > 0)
                def _():
                    store(s - 2, b).wait()

                parv = [
                    lax.shift_left(
                        lax.bitwise_and(idx_v[s, pl.ds(16 * m, 16)], 1), 6
                    )
                    for m in range(8)
                ]

                @plsc.parallel_loop(0, EMB_SIZE)
                def reorder(e):
                    for m in range(8):
                        val = plsc.load_gather(gbuf[b], [rowv[m], parv[m] + e])
                        obuf[b][e, pl.ds(16 * m, 16)] = val * SCALE

                @pl.when(r < n_outer - 1)
                def _():
                    gather(s + 2, b).start()

                store(s, b).start()

        for b in range(2):
            store(n_outer * 2 - 2 + b, b).wait()

    return sc_gather


@jax.jit
def kernel(tokens, emb_table):
    bsz, seq = tokens.shape
    packed = emb_table.reshape(VOCAB // 2, 2 * EMB_SIZE)
    tok_t = tokens.astype(jnp.int32).T
    out_phys = _make_gather_kernel(bsz, seq)(packed, tok_t)
    return jnp.transpose(out_phys, (2, 0, 1))
